# Initial kernel scaffold; baseline (speedup 1.0000x reference)
import functools

import jax
import jax.numpy as jnp
from jax import lax
from jax.experimental import pallas as pl
from jax.experimental.pallas import tpu as pltpu
from jax.experimental.pallas import tpu_sc as plsc

NUM_EMBEDDINGS = 1000000
D = 64
B = 16384
K = 50

NC = 2
NS = 16
NW = NC * NS
LANES = 16
DV = D // LANES

C = 2
CK = C * K
ROWS = B // C
CPW = ROWS // NW


def _bag_body(idx_hbm, sc_hbm, w_hbm, out_hbm, idx_v, sc_v, rows_v, out_v, sem):
    wid = lax.axis_index("s") * NC + lax.axis_index("c")
    base = wid * CPW

    def chunk(j, carry):
        row = base + j
        pltpu.sync_copy(idx_hbm.at[row], idx_v)
        pltpu.sync_copy(sc_hbm.at[row], sc_v)
        pltpu.async_copy(w_hbm.at[idx_v], rows_v, sem).wait()
        for c in range(C):
            accs = [jnp.zeros((LANES,), jnp.float32) for _ in range(DV)]
            for k in range(K):
                r = c * K + k
                s = plsc.load_gather(sc_v, [jnp.full((LANES,), r, jnp.int32)])
                for d in range(DV):
                    accs[d] = accs[d] + rows_v[r, pl.ds(d * LANES, LANES)] * s
            for d in range(DV):
                out_v[pl.ds(c * D + d * LANES, LANES)] = accs[d]
        pltpu.sync_copy(out_v, out_hbm.at[row])
        return carry

    lax.fori_loop(0, CPW, chunk, 0)


@jax.jit
def _run(idx2, sc2, weight):
    mesh = plsc.VectorSubcoreMesh(core_axis_name="c", subcore_axis_name="s")
    return pl.kernel(
        _bag_body,
        out_type=jax.ShapeDtypeStruct((ROWS, C * D), jnp.float32),
        mesh=mesh,
        scratch_types=[
            pltpu.VMEM((CK,), jnp.int32),
            pltpu.VMEM((CK,), jnp.float32),
            pltpu.VMEM((CK, D), jnp.float32),
            pltpu.VMEM((C * D,), jnp.float32),
            pltpu.SemaphoreType.DMA,
        ],
    )(idx2, sc2, weight)


def kernel(indices, scores, weight):
    idx2 = indices.astype(jnp.int32).reshape(ROWS, CK)
    sc2 = scores.reshape(ROWS, CK)
    out = _run(idx2, sc2, weight)
    return out.reshape(B, D)


# speedup vs baseline: 2.2065x; 2.2065x over previous
import jax
import jax.numpy as jnp
from jax import lax
from jax.experimental import pallas as pl
from jax.experimental.pallas import tpu as pltpu
from jax.experimental.pallas import tpu_sc as plsc

NUM_EMBEDDINGS = 1000000
D = 64
B = 16384
K = 50

NC = 2
NS = 16
NW = NC * NS
LANES = 16
DV = D // LANES

C = 8
CK = C * K
G = 80
NG = CK // G
ROWS = B // C
CPW = ROWS // NW


def _bag_body(idx_hbm, sc_hbm, w_hbm, out_hbm, idx_v, sc_v, rows_v, out_v, sem):
    wid = lax.axis_index("s") * NC + lax.axis_index("c")
    base = wid * CPW

    def chunk(j, carry):
        row = base + j
        pltpu.sync_copy(idx_hbm.at[row], idx_v)
        pltpu.sync_copy(sc_hbm.at[row], sc_v)
        descs = [
            pltpu.async_copy(
                w_hbm.at[idx_v.at[i]], rows_v.at[pl.ds(i * G, G)], sem)
            for i in range(NG)
        ]
        for d_ in descs:
            d_.wait()

        def bag(c, carry2):
            accs = [jnp.zeros((LANES,), jnp.float32) for _ in range(DV)]
            rbase = c * K
            for k in range(K):
                r = rbase + k
                s = plsc.load_gather(sc_v, [jnp.full((LANES,), 0, jnp.int32) + r])
                for d in range(DV):
                    accs[d] = accs[d] + rows_v[r, pl.ds(d * LANES, LANES)] * s
            obase = pl.multiple_of(c * D, D)
            for d in range(DV):
                out_v[pl.ds(obase + d * LANES, LANES)] = accs[d]
            return carry2

        lax.fori_loop(0, C, bag, 0)
        pltpu.sync_copy(out_v, out_hbm.at[row])
        return carry

    lax.fori_loop(0, CPW, chunk, 0)


@jax.jit
def _run(idx3, sc2, weight):
    mesh = plsc.VectorSubcoreMesh(core_axis_name="c", subcore_axis_name="s")
    return pl.kernel(
        _bag_body,
        out_type=jax.ShapeDtypeStruct((ROWS, C * D), jnp.float32),
        mesh=mesh,
        compiler_params=pltpu.CompilerParams(
            needs_layout_passes=False, use_tc_tiling_on_sc=False),
        scratch_types=[
            pltpu.VMEM((NG, G), jnp.int32),
            pltpu.VMEM((CK,), jnp.float32),
            pltpu.VMEM((CK, D), jnp.float32),
            pltpu.VMEM((C * D,), jnp.float32),
            pltpu.SemaphoreType.DMA,
        ],
    )(idx3, sc2, weight)


def kernel(indices, scores, weight):
    idx3 = indices.astype(jnp.int32).reshape(ROWS, NG, G)
    sc2 = scores.reshape(ROWS, CK)
    out = _run(idx3, sc2, weight)
    return out.reshape(B, D)


# speedup vs baseline: 2.4443x; 1.1078x over previous
import jax
import jax.numpy as jnp
from jax import lax
from jax.experimental import pallas as pl
from jax.experimental.pallas import tpu as pltpu
from jax.experimental.pallas import tpu_sc as plsc

NUM_EMBEDDINGS = 1000000
D = 64
B = 16384
K = 50

NC = 2
NS = 16
NW = NC * NS
LANES = 16
DV = D // LANES

C = 8
CK = C * K
G = 80
NG = CK // G
ROWS = B // C
CPW = ROWS // NW


def _bag_body(idx_hbm, sc_hbm, w_hbm, out_hbm,
              idx_v, sc_v, rows_v, out_v, sem0, sem1):
    wid = lax.axis_index("s") * NC + lax.axis_index("c")
    base = wid * CPW
    sems = (sem0, sem1)

    def fetch(j, b):
        pltpu.sync_copy(idx_hbm.at[base + j], idx_v.at[b])
        pltpu.sync_copy(sc_hbm.at[base + j], sc_v.at[b])

    def fire(b):
        for i in range(NG):
            pltpu.async_copy(
                w_hbm.at[idx_v.at[b, i]],
                rows_v.at[b, pl.ds(i * G, G)], sems[b])

    def drain(b):
        pltpu.make_async_copy(
            w_hbm.at[pl.ds(0, CK)], rows_v.at[b], sems[b]).wait()

    def compute(b, j):
        def bag(c, carry2):
            accs = [jnp.zeros((LANES,), jnp.float32) for _ in range(DV)]
            rbase = c * K
            for k in range(K):
                r = rbase + k
                s = plsc.load_gather(
                    sc_v.at[b], [jnp.full((LANES,), 0, jnp.int32) + r])
                for d in range(DV):
                    accs[d] = accs[d] + rows_v[b, r, pl.ds(d * LANES, LANES)] * s
            obase = pl.multiple_of(c * D, D)
            for d in range(DV):
                out_v[b, pl.ds(obase + d * LANES, LANES)] = accs[d]
            return carry2

        lax.fori_loop(0, C, bag, 0)
        pltpu.sync_copy(out_v.at[b], out_hbm.at[base + j])

    fetch(0, 0)
    fire(0)
    fetch(1, 1)

    def step(j2, carry):
        for half in range(2):
            j = 2 * j2 + half
            b, nb = half, 1 - half

            drain(b)

            @pl.when(j + 1 < CPW)
            def _():
                fire(nb)

            compute(b, j)

            @pl.when(j + 2 < CPW)
            def _():
                fetch(j + 2, b)
        return carry

    lax.fori_loop(0, CPW // 2, step, 0)


@jax.jit
def _run(idx3, sc2, weight):
    mesh = plsc.VectorSubcoreMesh(core_axis_name="c", subcore_axis_name="s")
    return pl.kernel(
        _bag_body,
        out_type=jax.ShapeDtypeStruct((ROWS, C * D), jnp.float32),
        mesh=mesh,
        compiler_params=pltpu.CompilerParams(
            needs_layout_passes=False, use_tc_tiling_on_sc=False),
        scratch_types=[
            pltpu.VMEM((2, NG, G), jnp.int32),
            pltpu.VMEM((2, CK), jnp.float32),
            pltpu.VMEM((2, CK, D), jnp.float32),
            pltpu.VMEM((2, C * D), jnp.float32),
            pltpu.SemaphoreType.DMA,
            pltpu.SemaphoreType.DMA,
        ],
    )(idx3, sc2, weight)


def kernel(indices, scores, weight):
    idx3 = indices.astype(jnp.int32).reshape(ROWS, NG, G)
    sc2 = scores.reshape(ROWS, CK)
    out = _run(idx3, sc2, weight)
    return out.reshape(B, D)


# speedup vs baseline: 2.5852x; 1.0577x over previous
import jax
import jax.numpy as jnp
from jax import lax
from jax.experimental import pallas as pl
from jax.experimental.pallas import tpu as pltpu
from jax.experimental.pallas import tpu_sc as plsc

NUM_EMBEDDINGS = 1000000
D = 64
B = 16384
K = 50

NC = 2
NS = 16
NW = NC * NS
LANES = 16
DV = D // LANES

C = 8
CK = C * K
KP = 64
G = 80
NG = CK // G
ROWS = B // C
CPW = ROWS // NW


def _bag_body(idx_hbm, sc_hbm, w_hbm, out_hbm,
              idx_v, sc_v, rows_v, out_v, sem0, sem1):
    wid = lax.axis_index("s") * NC + lax.axis_index("c")
    base = wid * CPW
    sems = (sem0, sem1)

    def fetch(j, b):
        pltpu.sync_copy(idx_hbm.at[base + j], idx_v.at[b])
        pltpu.sync_copy(sc_hbm.at[base + j], sc_v.at[b])

    lane_ids = [jnp.full((LANES, 1), lane, jnp.int32) for lane in range(LANES)]
    _gdn = lax.GatherDimensionNumbers(
        offset_dims=(), collapsed_slice_dims=(0,), start_index_map=(0,))

    def _splat(sv, lane):
        return lax.gather(sv, lane_ids[lane], dimension_numbers=_gdn,
                          slice_sizes=(1,),
                          mode=lax.GatherScatterMode.PROMISE_IN_BOUNDS)

    def fire(b):
        for i in range(NG):
            pltpu.async_copy(
                w_hbm.at[idx_v.at[b, i]],
                rows_v.at[b, pl.ds(i * G, G)], sems[b])

    def drain(b):
        pltpu.make_async_copy(
            w_hbm.at[pl.ds(0, CK)], rows_v.at[b], sems[b]).wait()

    def compute(b, j):
        def bag(c, carry2):
            accs = [jnp.zeros((LANES,), jnp.float32) for _ in range(DV)]
            rbase = c * K
            sbase = pl.multiple_of(c * KP, KP)
            svs = [sc_v[b, pl.ds(sbase + g * LANES, LANES)]
                   for g in range(KP // LANES)]
            for k in range(K):
                r = rbase + k
                s = _splat(svs[k // LANES], k % LANES)
                for d in range(DV):
                    accs[d] = accs[d] + rows_v[b, r, pl.ds(d * LANES, LANES)] * s
            obase = pl.multiple_of(c * D, D)
            for d in range(DV):
                out_v[b, pl.ds(obase + d * LANES, LANES)] = accs[d]
            return carry2

        lax.fori_loop(0, C, bag, 0)
        pltpu.sync_copy(out_v.at[b], out_hbm.at[base + j])

    fetch(0, 0)
    fire(0)
    fetch(1, 1)

    def step(j2, carry):
        for half in range(2):
            j = 2 * j2 + half
            b, nb = half, 1 - half

            drain(b)

            @pl.when(j + 1 < CPW)
            def _():
                fire(nb)

            compute(b, j)

            @pl.when(j + 2 < CPW)
            def _():
                fetch(j + 2, b)
        return carry

    lax.fori_loop(0, CPW // 2, step, 0)


@jax.jit
def _run(idx3, sc2, weight):
    mesh = plsc.VectorSubcoreMesh(core_axis_name="c", subcore_axis_name="s")
    return pl.kernel(
        _bag_body,
        out_type=jax.ShapeDtypeStruct((ROWS, C * D), jnp.float32),
        mesh=mesh,
        compiler_params=pltpu.CompilerParams(
            needs_layout_passes=False, use_tc_tiling_on_sc=False),
        scratch_types=[
            pltpu.VMEM((2, NG, G), jnp.int32),
            pltpu.VMEM((2, C * KP), jnp.float32),
            pltpu.VMEM((2, CK, D), jnp.float32),
            pltpu.VMEM((2, C * D), jnp.float32),
            pltpu.SemaphoreType.DMA,
            pltpu.SemaphoreType.DMA,
        ],
    )(idx3, sc2, weight)


def kernel(indices, scores, weight):
    idx3 = indices.astype(jnp.int32).reshape(ROWS, NG, G)
    scp = jnp.pad(scores, ((0, 0), (0, KP - K)))
    sc2 = scp.reshape(ROWS, C * KP)
    out = _run(idx3, sc2, weight)
    return out.reshape(B, D)
